# baseline (speedup 1.0000x reference)
import functools

import jax
import jax.numpy as jnp
from jax import lax
from jax.experimental import pallas as pl
from jax.experimental.pallas import tpu as pltpu
from jax.experimental.pallas import tpu_sc as plsc

BATCH = 4096
SEQ = 200
NNUC = 4
DIM = 128

NW = 32
NSUB = 16
ROWS = BATCH * SEQ
RPW = ROWS // NW
UNIT = 128
NUNITS = RPW // UNIT
NRING = 4
LANES = 16
CROWS = SEQ * NNUC
CPT = CROWS // NSUB
PPT = CPT // NNUC + 1


def _sc_body(x_hbm, nuc_hbm, pos_hbm, out_hbm, comb_sh, x_v, nuc_v, pos_v,
             *rest):
    rings = rest[:NRING]
    gsems = rest[NRING:2 * NRING]
    ssems = rest[2 * NRING:]

    cid = lax.axis_index("c")
    sid = lax.axis_index("s")
    wid = sid * 2 + cid
    base0 = wid * RPW
    lane = lax.iota(jnp.int32, LANES)

    pltpu.sync_copy(x_hbm.at[pl.ds(base0, RPW)], x_v)

    crow0 = sid * CPT
    lrow0 = crow0 // NNUC
    pltpu.sync_copy(nuc_hbm, nuc_v)
    pltpu.sync_copy(pos_hbm.at[pl.ds(lrow0 * DIM, PPT * DIM)], pos_v)

    def build_row(r, carry):
        crow = crow0 + r
        l_loc = crow // NNUC - lrow0
        n = crow % NNUC
        for g in range(DIM // LANES):
            rings[0][r, pl.ds(g * LANES, LANES)] = (
                pos_v[pl.ds(l_loc * DIM + g * LANES, LANES)]
                + nuc_v[n, pl.ds(g * LANES, LANES)]
            )
        return carry

    lax.fori_loop(0, CPT, build_row, 0)
    pltpu.sync_copy(rings[0].at[pl.ds(0, CPT)], comb_sh.at[pl.ds(crow0, CPT)])
    plsc.subcore_barrier()

    def compute_idx(k):
        for g in range(UNIT // LANES):
            off = k * UNIT + g * LANES
            r = (base0 + off) + lane
            x_v[pl.ds(off, LANES)] = (r % SEQ) * 4 + x_v[pl.ds(off, LANES)]

    def start_gather(k, u):
        compute_idx(k)
        pltpu.async_copy(
            comb_sh.at[x_v.at[pl.ds(k * UNIT, UNIT)]], rings[u], gsems[u]
        )

    def start_scatter(k, u):
        pltpu.async_copy(
            rings[u], out_hbm.at[pl.ds(base0 + k * UNIT, UNIT)], ssems[u]
        )

    def wait_gather(u):
        pltpu.make_async_copy(out_hbm.at[pl.ds(0, UNIT)], rings[u],
                              gsems[u]).wait()

    def wait_scatter(u):
        pltpu.make_async_copy(rings[u], out_hbm.at[pl.ds(0, UNIT)],
                              ssems[u]).wait()

    for u in range(NRING):
        start_gather(u, u)

    def body(j, carry):
        k = j * NRING
        for u in range(NRING):
            wait_gather(u)
            start_scatter(k + u, u)
        for u in range(NRING):
            wait_scatter(u)
            start_gather(k + NRING + u, u)
        return carry

    lax.fori_loop(0, NUNITS // NRING - 1, body, 0)

    for u in range(NRING):
        wait_gather(u)
        start_scatter(NUNITS - NRING + u, u)
    for u in range(NRING):
        wait_scatter(u)


def kernel(X, nucleo_emb, pos_emb):
    x_flat = X.reshape(ROWS)

    mesh = plsc.VectorSubcoreMesh(core_axis_name="c", subcore_axis_name="s")
    sc_embed = functools.partial(
        pl.kernel,
        mesh=mesh,
        out_type=jax.ShapeDtypeStruct((ROWS, DIM), jnp.float32),
        scratch_types=(
            [pltpu.VMEM_SHARED((CROWS, DIM), jnp.float32),
             pltpu.VMEM((RPW,), jnp.int32),
             pltpu.VMEM((NNUC, DIM), jnp.float32),
             pltpu.VMEM((PPT * DIM,), jnp.float32)]
            + [pltpu.VMEM((UNIT, DIM), jnp.float32)] * NRING
            + [pltpu.SemaphoreType.DMA] * (2 * NRING)
        ),
    )(_sc_body)

    out = sc_embed(x_flat, nucleo_emb, pos_emb.reshape(SEQ * DIM))
    return out.reshape(BATCH, SEQ, DIM)
